# baseline (speedup 1.0000x reference)
import functools

import jax
import jax.numpy as jnp
from jax import lax
from jax.experimental import pallas as pl
from jax.experimental.pallas import tpu as pltpu
from jax.experimental.pallas import tpu_sc as plsc

_VOCAB = 1_000_000
_DIM = 32
_BATCH = 16384
_NC = 2
_NS = 16
_NW = _NC * _NS
_BPW = _BATCH // _NW
_GRP = _BPW // 16


def _glove_body(i_hbm, j_hbm, wi_hbm, wj_hbm, bi_hbm, bj_hbm, out_hbm,
                ii_v, jj_v, wi_v, wj_v, bi_v, bj_v, out_v, sem):
    wid = lax.axis_index("s") * _NC + lax.axis_index("c")
    base = wid * _BPW

    pltpu.sync_copy(i_hbm.at[pl.ds(base, _BPW)], ii_v)
    pltpu.sync_copy(j_hbm.at[pl.ds(base, _BPW)], jj_v)

    c1 = pltpu.async_copy(wi_hbm.at[ii_v], wi_v, sem)
    c2 = pltpu.async_copy(wj_hbm.at[jj_v], wj_v, sem)
    c3 = pltpu.async_copy(bi_hbm.at[ii_v], bi_v, sem)
    c4 = pltpu.async_copy(bj_hbm.at[jj_v], bj_v, sem)
    c1.wait()
    c2.wait()
    c3.wait()
    c4.wait()

    def group(g, carry):
        rows = g * 16 + lax.iota(jnp.int32, 16)
        acc = bi_v[pl.ds(g * 16, 16)] + bj_v[pl.ds(g * 16, 16)]
        for d in range(_DIM):
            col = jnp.full((16,), d, dtype=jnp.int32)
            a = plsc.load_gather(wi_v, [rows, col])
            b = plsc.load_gather(wj_v, [rows, col])
            acc = acc + a * b
        out_v[pl.ds(g * 16, 16)] = acc
        return carry

    lax.fori_loop(0, _GRP, group, 0)
    pltpu.sync_copy(out_v, out_hbm.at[pl.ds(base, _BPW)])


@jax.jit
def _glove_call(i32, j32, wi, wj, bi_flat, bj_flat):
    mesh = plsc.VectorSubcoreMesh(core_axis_name="c", subcore_axis_name="s")
    run = pl.kernel(
        _glove_body,
        mesh=mesh,
        compiler_params=pltpu.CompilerParams(
            needs_layout_passes=False, use_tc_tiling_on_sc=False
        ),
        out_type=jax.ShapeDtypeStruct((_BATCH,), jnp.float32),
        scratch_types=[
            pltpu.VMEM((_BPW,), jnp.int32),
            pltpu.VMEM((_BPW,), jnp.int32),
            pltpu.VMEM((_BPW, _DIM), jnp.float32),
            pltpu.VMEM((_BPW, _DIM), jnp.float32),
            pltpu.VMEM((_BPW,), jnp.float32),
            pltpu.VMEM((_BPW,), jnp.float32),
            pltpu.VMEM((_BPW,), jnp.float32),
            pltpu.SemaphoreType.DMA,
        ],
    )
    return run(i32, j32, wi, wj, bi_flat, bj_flat)


def kernel(i, j, wi, wj, bi, bj):
    i32 = i.astype(jnp.int32)
    j32 = j.astype(jnp.int32)
    out = _glove_call(i32, j32, wi, wj, bi.reshape(-1), bj.reshape(-1))
    return out.reshape(_BATCH, 1)
